# Initial kernel scaffold; baseline (speedup 1.0000x reference)
import jax
import jax.numpy as jnp
from jax.experimental import pallas as pl


def kernel(caption_tknID, img_fc, emb_table, W1, b1):
    raise NotImplementedError("write your pallas kernel here")


# speedup vs baseline: 1.1459x; 1.1459x over previous
import functools

import jax
import jax.numpy as jnp
from jax import lax
from jax.experimental import pallas as pl
from jax.experimental.pallas import tpu as pltpu
from jax.experimental.pallas import tpu_sc as plsc


def _proj_body(emb_ref, w_ref, b_ref, out_ref):
    out_ref[...] = lax.dot_general(
        emb_ref[...], w_ref[...],
        dimension_numbers=(((1,), (1,)), ((), ())),
        preferred_element_type=jnp.float32,
    ) + b_ref[...]


def _project_table(emb_table, W1, b1):
    V, D = emb_table.shape
    return pl.pallas_call(
        _proj_body,
        out_shape=jax.ShapeDtypeStruct((V, D), jnp.float32),
    )(emb_table, W1, b1.reshape(1, D))


_CHUNK = 64
_D = 512


def _make_sc_gather(num_tokens, tokens_per_row):
    info = plsc.get_sparse_core_info()
    NC, NS = info.num_cores, info.num_subcores
    NW = NC * NS
    toks_per_w = num_tokens // NW
    chunks_per_w = toks_per_w // _CHUNK
    mesh = plsc.VectorSubcoreMesh(core_axis_name="c", subcore_axis_name="s")

    @functools.partial(
        pl.kernel,
        mesh=mesh,
        out_type=jax.ShapeDtypeStruct((num_tokens, 2 * _D), jnp.float32),
        scratch_types=[
            pltpu.VMEM((chunks_per_w, _CHUNK), jnp.int32),
            pltpu.VMEM((chunks_per_w, _CHUNK), jnp.int32),
            pltpu.VMEM((_CHUNK, _D), jnp.float32),
            pltpu.VMEM((_CHUNK, _D), jnp.float32),
            pltpu.SemaphoreType.DMA,
            pltpu.SemaphoreType.DMA,
        ],
    )
    def sc_kernel(proj_hbm, idx_hbm, rep_hbm, img_hbm, out_hbm,
                  idx_v, rep_v, wbuf, ibuf, wsem, isem):
        wid = lax.axis_index("s") * NC + lax.axis_index("c")
        t0 = wid * toks_per_w

        pltpu.sync_copy(idx_hbm.at[wid], idx_v)
        pltpu.sync_copy(rep_hbm.at[wid], rep_v)

        wcopy = pltpu.async_copy(proj_hbm.at[idx_v.at[0]], wbuf, wsem)
        icopy = pltpu.async_copy(img_hbm.at[rep_v.at[0]], ibuf, isem)
        for j in range(chunks_per_w):
            t = t0 + j * _CHUNK
            wcopy.wait()
            pltpu.sync_copy(wbuf, out_hbm.at[pl.ds(t, _CHUNK), pl.ds(0, _D)])
            if j + 1 < chunks_per_w:
                wcopy = pltpu.async_copy(proj_hbm.at[idx_v.at[j + 1]],
                                         wbuf, wsem)
            icopy.wait()
            pltpu.sync_copy(ibuf, out_hbm.at[pl.ds(t, _CHUNK), pl.ds(_D, _D)])
            if j + 1 < chunks_per_w:
                icopy = pltpu.async_copy(img_hbm.at[rep_v.at[j + 1]],
                                         ibuf, isem)

    return sc_kernel


def kernel(caption_tknID, img_fc, emb_table, W1, b1):
    B, L = caption_tknID.shape
    num_tokens = B * L
    proj = _project_table(emb_table, W1, b1)
    info = plsc.get_sparse_core_info()
    nw = info.num_cores * info.num_subcores
    shape3d = (nw, num_tokens // (nw * _CHUNK), _CHUNK)
    idx3d = caption_tknID.astype(jnp.int32).reshape(shape3d)
    rep3d = jnp.reshape(
        jnp.repeat(jnp.arange(B, dtype=jnp.int32), L), shape3d)
    out_flat = _make_sc_gather(num_tokens, L)(proj, idx3d, rep3d, img_fc)
    return out_flat.reshape(B, L, 2 * _D)
